# Initial kernel scaffold; baseline (speedup 1.0000x reference)
import jax
import jax.numpy as jnp
from jax.experimental import pallas as pl


def kernel(x, edge_index, W0, b0, W1, b1):
    raise NotImplementedError("write your pallas kernel here")


# speedup vs baseline: 13.0566x; 13.0566x over previous
import functools

import jax
import jax.numpy as jnp
from jax import lax
from jax.experimental import pallas as pl
from jax.experimental.pallas import tpu as pltpu
from jax.experimental.pallas import tpu_sc as plsc

NC = 2
NS = 16
NW = NC * NS
CHUNK = 128
LANES = 16


def _sc_mesh():
  return plsc.VectorSubcoreMesh(core_axis_name="c", subcore_axis_name="s")


@functools.partial(jax.jit, static_argnums=(1, 2))
def _sc_deg(dst, n_pad, e_pad):
  epw = e_pad // NW
  cpw = epw // CHUNK
  rpt = n_pad // NS

  @functools.partial(
      pl.kernel,
      mesh=_sc_mesh(),
      compiler_params=pltpu.CompilerParams(use_tc_tiling_on_sc=False),
      out_type=jax.ShapeDtypeStruct((NC * n_pad,), jnp.float32),
      scratch_types=[
          pltpu.VMEM((CHUNK,), jnp.int32),
          pltpu.VMEM((CHUNK,), jnp.float32),
          pltpu.VMEM((rpt,), jnp.float32),
          pltpu.VMEM_SHARED((n_pad,), jnp.float32),
      ],
  )
  def deg_kernel(dst_hbm, out_hbm, idx_v, ones_v, bounce_v, acc_sh):
    c = lax.axis_index("c")
    s = lax.axis_index("s")
    wid = c * NS + s

    def init_ones(i, _):
      ones_v[pl.ds(i * LANES, LANES)] = jnp.full((LANES,), 1.0, jnp.float32)
      return 0
    lax.fori_loop(0, CHUNK // LANES, init_ones, 0)

    def init_zero(i, _):
      bounce_v[pl.ds(i * LANES, LANES)] = jnp.zeros((LANES,), jnp.float32)
      return 0
    lax.fori_loop(0, rpt // LANES, init_zero, 0)

    pltpu.sync_copy(bounce_v, acc_sh.at[pl.ds(s * rpt, rpt)])
    plsc.subcore_barrier()

    base = wid * epw

    def body(i, _):
      pltpu.sync_copy(dst_hbm.at[pl.ds(base + i * CHUNK, CHUNK)], idx_v)
      pltpu.sync_copy(ones_v, acc_sh.at[idx_v], add=True)
      return 0
    lax.fori_loop(0, cpw, body, 0)

    plsc.subcore_barrier()
    pltpu.sync_copy(acc_sh.at[pl.ds(s * rpt, rpt)], bounce_v)
    pltpu.sync_copy(bounce_v, out_hbm.at[pl.ds(c * n_pad + s * rpt, rpt)])

  return deg_kernel(dst)


@functools.partial(jax.jit, static_argnums=(3, 4, 5))
def _sc_agg(table, src, dst, n_pad, e_pad, h):
  epw = e_pad // NW
  cpw = epw // CHUNK
  rpt = n_pad // NS
  zr = 128
  nz = rpt // zr

  @functools.partial(
      pl.kernel,
      mesh=_sc_mesh(),
      compiler_params=pltpu.CompilerParams(use_tc_tiling_on_sc=False),
      out_type=jax.ShapeDtypeStruct((NC * n_pad, h), jnp.float32),
      scratch_types=[
          pltpu.VMEM((CHUNK,), jnp.int32),
          pltpu.VMEM((CHUNK,), jnp.int32),
          pltpu.VMEM((CHUNK, h), jnp.float32),
          pltpu.VMEM((zr, h), jnp.float32),
          pltpu.VMEM_SHARED((n_pad, h), jnp.float32),
          pltpu.SemaphoreType.DMA,
      ],
  )
  def agg_kernel(tbl_hbm, src_hbm, dst_hbm, out_hbm,
                 sidx_v, didx_v, rows_v, bounce_v, acc_sh, sem):
    c = lax.axis_index("c")
    s = lax.axis_index("s")
    wid = c * NS + s

    def init_zero(i, _):
      r = i // (h // LANES)
      j = i % (h // LANES)
      bounce_v[r, pl.ds(j * LANES, LANES)] = jnp.zeros((LANES,), jnp.float32)
      return 0
    lax.fori_loop(0, zr * (h // LANES), init_zero, 0)

    def zero_acc(k, _):
      pltpu.sync_copy(bounce_v, acc_sh.at[pl.ds(s * rpt + k * zr, zr)])
      return 0
    lax.fori_loop(0, nz, zero_acc, 0)
    plsc.subcore_barrier()

    base = wid * epw

    def body(i, _):
      pltpu.sync_copy(src_hbm.at[pl.ds(base + i * CHUNK, CHUNK)], sidx_v)
      pltpu.sync_copy(dst_hbm.at[pl.ds(base + i * CHUNK, CHUNK)], didx_v)
      pltpu.async_copy(tbl_hbm.at[sidx_v], rows_v, sem).wait()
      pltpu.sync_copy(rows_v, acc_sh.at[didx_v], add=True)
      return 0
    lax.fori_loop(0, cpw, body, 0)

    plsc.subcore_barrier()

    def writeout(k, _):
      pltpu.sync_copy(acc_sh.at[pl.ds(s * rpt + k * zr, zr)], bounce_v)
      pltpu.sync_copy(
          bounce_v, out_hbm.at[pl.ds(c * n_pad + s * rpt + k * zr, zr)])
      return 0
    lax.fori_loop(0, nz, writeout, 0)

  return agg_kernel(table, src, dst)


def _tc_mm1(x_p, w0, degp, n_pad, bm=1024):
  nfeat = x_p.shape[1]
  h0 = w0.shape[1]

  def body(x_ref, w_ref, dg_ref, m_ref, dv_ref):
    dsum = dg_ref[0:1, :] + dg_ref[1:2, :] + 1.0
    dcol = jnp.transpose(lax.rsqrt(dsum), (1, 0))
    acc = jnp.dot(x_ref[...], w_ref[...], preferred_element_type=jnp.float32)
    m_ref[...] = acc * dcol
    dv_ref[...] = dcol

  return pl.pallas_call(
      body,
      grid=(n_pad // bm,),
      in_specs=[
          pl.BlockSpec((bm, nfeat), lambda i: (i, 0)),
          pl.BlockSpec((nfeat, h0), lambda i: (0, 0)),
          pl.BlockSpec((NC, bm), lambda i: (0, i)),
      ],
      out_specs=[
          pl.BlockSpec((bm, h0), lambda i: (i, 0)),
          pl.BlockSpec((bm, 1), lambda i: (i, 0)),
      ],
      out_shape=[
          jax.ShapeDtypeStruct((n_pad, h0), jnp.float32),
          jax.ShapeDtypeStruct((n_pad, 1), jnp.float32),
      ],
  )(x_p, w0, degp)


def _tc_mid(p0, p1, m1s, dinv, b0r, w1, n_pad, bm=1024):
  h0 = m1s.shape[1]
  ncls = w1.shape[1]

  def body(p0_ref, p1_ref, m_ref, dv_ref, b_ref, w_ref, o_ref):
    t = (p0_ref[...] + p1_ref[...] + m_ref[...]) * dv_ref[...] + b_ref[...]
    h1 = jnp.maximum(t, 0.0)
    o_ref[...] = jnp.dot(
        h1, w_ref[...], preferred_element_type=jnp.float32) * dv_ref[...]

  return pl.pallas_call(
      body,
      grid=(n_pad // bm,),
      in_specs=[
          pl.BlockSpec((bm, h0), lambda i: (i, 0)),
          pl.BlockSpec((bm, h0), lambda i: (i, 0)),
          pl.BlockSpec((bm, h0), lambda i: (i, 0)),
          pl.BlockSpec((bm, 1), lambda i: (i, 0)),
          pl.BlockSpec((1, h0), lambda i: (0, 0)),
          pl.BlockSpec((h0, ncls), lambda i: (0, 0)),
      ],
      out_specs=pl.BlockSpec((bm, ncls), lambda i: (i, 0)),
      out_shape=jax.ShapeDtypeStruct((n_pad, ncls), jnp.float32),
  )(p0, p1, m1s, dinv, b0r, w1)


def _tc_out(q0, q1, m2s, dinv, b1r, n_pad, bm=1024):
  ncls = m2s.shape[1]

  def body(q0_ref, q1_ref, m_ref, dv_ref, b_ref, o_ref):
    t = (q0_ref[...] + q1_ref[...] + m_ref[...]) * dv_ref[...] + b_ref[...]
    z = jnp.maximum(t, 0.0)
    zmax = jnp.max(z, axis=1, keepdims=True)
    e = jnp.exp(z - zmax)
    ssum = jnp.sum(e, axis=1, keepdims=True)
    o_ref[...] = (z - zmax) - jnp.log(ssum)

  return pl.pallas_call(
      body,
      grid=(n_pad // bm,),
      in_specs=[
          pl.BlockSpec((bm, ncls), lambda i: (i, 0)),
          pl.BlockSpec((bm, ncls), lambda i: (i, 0)),
          pl.BlockSpec((bm, ncls), lambda i: (i, 0)),
          pl.BlockSpec((bm, 1), lambda i: (i, 0)),
          pl.BlockSpec((1, ncls), lambda i: (0, 0)),
      ],
      out_specs=pl.BlockSpec((bm, ncls), lambda i: (i, 0)),
      out_shape=jax.ShapeDtypeStruct((n_pad, ncls), jnp.float32),
  )(q0, q1, m2s, dinv, b1r)


def kernel(x, edge_index, W0, b0, W1, b1):
  n, nfeat = x.shape
  e = edge_index.shape[1]

  n_pad = -(-n // 2048) * 2048
  e_pad = -(-e // (NW * CHUNK)) * (NW * CHUNK)

  x_p = jnp.zeros((n_pad, nfeat), jnp.float32).at[:n].set(x)
  pad = jnp.full((e_pad - e,), n, jnp.int32)
  src = jnp.concatenate([edge_index[0], pad])
  dst = jnp.concatenate([edge_index[1], pad])

  degp = _sc_deg(dst, n_pad, e_pad).reshape(NC, n_pad)
  m1s, dinv = _tc_mm1(x_p, W0, degp, n_pad)

  p = _sc_agg(m1s, src, dst, n_pad, e_pad, W0.shape[1]).reshape(NC, n_pad, -1)
  m2s = _tc_mid(p[0], p[1], m1s, dinv, b0.reshape(1, -1), W1, n_pad)

  q = _sc_agg(m2s, src, dst, n_pad, e_pad, W1.shape[1]).reshape(NC, n_pad, -1)
  out = _tc_out(q[0], q[1], m2s, dinv, b1.reshape(1, -1), n_pad)
  return out[:n]


# speedup vs baseline: 15.6884x; 1.2016x over previous
import functools

import jax
import jax.numpy as jnp
from jax import lax
from jax.experimental import pallas as pl
from jax.experimental.pallas import tpu as pltpu
from jax.experimental.pallas import tpu_sc as plsc

NC = 2
NS = 16
NW = NC * NS
CHUNK = 128
LANES = 16


def _sc_mesh():
  return plsc.VectorSubcoreMesh(core_axis_name="c", subcore_axis_name="s")


@functools.partial(jax.jit, static_argnums=(1, 2, 3))
def _sc_deg(dst, n_pad, e_pad, cpw0):
  ct16 = e_pad // CHUNK // NS
  cpw1 = ct16 - cpw0
  cpw = max(cpw0, cpw1)
  rpt = n_pad // NS

  @functools.partial(
      pl.kernel,
      mesh=_sc_mesh(),
      compiler_params=pltpu.CompilerParams(use_tc_tiling_on_sc=False),
      out_type=jax.ShapeDtypeStruct((NC * n_pad,), jnp.float32),
      scratch_types=[
          pltpu.VMEM((cpw, CHUNK), jnp.int32),
          pltpu.VMEM((CHUNK,), jnp.float32),
          pltpu.VMEM((rpt,), jnp.float32),
          pltpu.VMEM_SHARED((n_pad,), jnp.float32),
          pltpu.SemaphoreType.DMA,
          pltpu.SemaphoreType.DMA,
      ],
  )
  def deg_kernel(dst_hbm, out_hbm, didx_v, ones_v, bounce_v, acc_sh,
                 sem_i, sem_s):
    c = lax.axis_index("c")
    s = lax.axis_index("s")
    cpw_c = jnp.where(c == 0, cpw0, cpw1)
    base = jnp.where(c == 0, s * cpw0, NS * cpw0 + s * cpw1) * CHUNK

    def ld(i, _):
      pltpu.async_copy(
          dst_hbm.at[pl.ds(base + i * CHUNK, CHUNK)], didx_v.at[i], sem_i)
      return 0
    lax.fori_loop(0, cpw_c, ld, 0)

    def init_ones(i, _):
      ones_v[pl.ds(i * LANES, LANES)] = jnp.full((LANES,), 1.0, jnp.float32)
      return 0
    lax.fori_loop(0, CHUNK // LANES, init_ones, 0)

    def init_zero(i, _):
      bounce_v[pl.ds(i * LANES, LANES)] = jnp.zeros((LANES,), jnp.float32)
      return 0
    lax.fori_loop(0, rpt // LANES, init_zero, 0)

    pltpu.sync_copy(bounce_v, acc_sh.at[pl.ds(s * rpt, rpt)])

    def ld_drain(i, _):
      pltpu.make_async_copy(
          dst_hbm.at[pl.ds(0, CHUNK)], didx_v.at[0], sem_i).wait()
      return 0
    lax.fori_loop(0, cpw_c, ld_drain, 0)
    plsc.subcore_barrier()

    def body(i, _):
      pltpu.async_copy(ones_v, acc_sh.at[didx_v.at[i]], sem_s, add=True)
      return 0
    lax.fori_loop(0, cpw_c, body, 0)

    def drain(i, _):
      pltpu.make_async_copy(ones_v, acc_sh.at[didx_v.at[0]], sem_s).wait()
      return 0
    lax.fori_loop(0, cpw_c, drain, 0)

    plsc.subcore_barrier()
    pltpu.sync_copy(acc_sh.at[pl.ds(s * rpt, rpt)], bounce_v)
    pltpu.sync_copy(bounce_v, out_hbm.at[pl.ds(c * n_pad + s * rpt, rpt)])

  return deg_kernel(dst)


RING = 5
PF = 3
RP = RING - PF
DI = 10
KI = 8


@functools.partial(jax.jit, static_argnums=(3, 4, 5, 6))
def _sc_agg(table, src, dst, n_pad, e_pad, h, cpw0):
  ct16 = e_pad // CHUNK // NS
  cpw1 = ct16 - cpw0
  rpt = n_pad // NS
  zr = 32
  nz = rpt // zr
  vmem_words = 2 * DI * CHUNK + RING * CHUNK * h + zr * h
  stage = (2 * n_pad * h + NS * vmem_words) <= 2_000_000

  scratch = [
      pltpu.VMEM((DI, CHUNK), jnp.int32),
      pltpu.VMEM((DI, CHUNK), jnp.int32),
      pltpu.VMEM((RING, CHUNK, h), jnp.float32),
      pltpu.VMEM((zr, h), jnp.float32),
      pltpu.VMEM_SHARED((n_pad, h), jnp.float32),
      pltpu.SemaphoreType.DMA((DI,)),
      pltpu.SemaphoreType.DMA((RING,)),
      pltpu.SemaphoreType.DMA((RING,)),
  ]
  if stage:
    scratch.append(pltpu.VMEM_SHARED((n_pad, h), jnp.float32))

  @functools.partial(
      pl.kernel,
      mesh=_sc_mesh(),
      compiler_params=pltpu.CompilerParams(use_tc_tiling_on_sc=False),
      out_type=jax.ShapeDtypeStruct((NC * n_pad, h), jnp.float32),
      scratch_types=scratch,
  )
  def agg_kernel(tbl_hbm, src_hbm, dst_hbm, out_hbm,
                 sidx_v, didx_v, rows_v, bounce_v, acc_sh,
                 sem_i, sem_g, sem_s, *maybe_tbl):
    tbl_ref = maybe_tbl[0] if stage else tbl_hbm
    c = lax.axis_index("c")
    s = lax.axis_index("s")
    cpw_c = jnp.where(c == 0, cpw0, cpw1)
    base = jnp.where(c == 0, s * cpw0, NS * cpw0 + s * cpw1) * CHUNK

    def ld(i, sl):
      pltpu.async_copy(
          src_hbm.at[pl.ds(base + i * CHUNK, CHUNK)], sidx_v.at[sl],
          sem_i.at[sl])
      pltpu.async_copy(
          dst_hbm.at[pl.ds(base + i * CHUNK, CHUNK)], didx_v.at[sl],
          sem_i.at[sl])

    def iwait(sl):
      pltpu.make_async_copy(
          src_hbm.at[pl.ds(0, CHUNK)], sidx_v.at[0], sem_i.at[sl]).wait()
      pltpu.make_async_copy(
          src_hbm.at[pl.ds(0, CHUNK)], didx_v.at[0], sem_i.at[sl]).wait()

    def gather(i, b, sl):
      del i
      pltpu.async_copy(tbl_ref.at[sidx_v.at[sl]], rows_v.at[b], sem_g.at[b])

    def gwait(b):
      pltpu.make_async_copy(
          tbl_ref.at[sidx_v.at[0]], rows_v.at[b], sem_g.at[b]).wait()

    def scatter(i, b, sl):
      del i
      pltpu.async_copy(
          rows_v.at[b], acc_sh.at[didx_v.at[sl]], sem_s.at[b], add=True)

    def swait(b):
      pltpu.make_async_copy(
          rows_v.at[b], acc_sh.at[didx_v.at[0]], sem_s.at[b]).wait()

    @pl.when(cpw_c > 0)
    def _work():
      for t in range(KI):
        ld(t, t)

      if stage:
        def stage_tbl(k, _):
          pltpu.sync_copy(
              tbl_hbm.at[pl.ds(s * rpt + k * zr, zr)],
              maybe_tbl[0].at[pl.ds(s * rpt + k * zr, zr)])
          return 0
        lax.fori_loop(0, nz, stage_tbl, 0)

      def init_zero(i, _):
        r = i // (h // LANES)
        j = i % (h // LANES)
        bounce_v[r, pl.ds(j * LANES, LANES)] = jnp.zeros(
            (LANES,), jnp.float32)
        return 0
      lax.fori_loop(0, zr * (h // LANES), init_zero, 0)

      def zero_acc(k, _):
        pltpu.sync_copy(bounce_v, acc_sh.at[pl.ds(s * rpt + k * zr, zr)])
        return 0
      lax.fori_loop(0, nz, zero_acc, 0)
      plsc.subcore_barrier()

      for t in range(PF):
        iwait(t)
        gather(t, t % RING, t)

      def handler(ibase, jj, t, load_ok, gather_ok, guard):
        i = ibase + t
        b = t % RING
        bp = (b + PF) % RING
        gwait(b)
        scatter(i, b, t)
        if guard:
          @pl.when(jj > 0)
          def _():
            swait(bp)
        else:
          swait(bp)
        if load_ok:
          ld(i + KI, (t + KI) % DI)
        if gather_ok:
          iwait((t + PF) % DI)
          gather(i + PF, bp, (t + PF) % DI)

      def group(jj, _):
        ibase = jj * DI
        for t in range(DI):
          handler(ibase, jj, t, True, True, t < RP)
        return 0
      lax.fori_loop(0, cpw_c // DI - 1, group, 0)

      ibase = cpw_c - DI
      for t in range(DI):
        handler(ibase, 1, t, t + KI < DI, t + PF < DI, False)
      for t in range(DI - RP, DI):
        swait(t % RING)

      plsc.subcore_barrier()

      def writeout(k, _):
        pltpu.sync_copy(acc_sh.at[pl.ds(s * rpt + k * zr, zr)], bounce_v)
        pltpu.sync_copy(
            bounce_v, out_hbm.at[pl.ds(c * n_pad + s * rpt + k * zr, zr)])
        return 0
      lax.fori_loop(0, nz, writeout, 0)

  return agg_kernel(table, src, dst)


def _tc_mm1(x_p, w0, degp, n_pad, bm=1024):
  nfeat = x_p.shape[1]
  h0 = w0.shape[1]

  def body(x_ref, w_ref, dg_ref, m_ref, dv_ref):
    dsum = dg_ref[0:1, :] + dg_ref[1:2, :] + 1.0
    dcol = jnp.transpose(lax.rsqrt(dsum), (1, 0))
    acc = jnp.dot(x_ref[...], w_ref[...], preferred_element_type=jnp.float32)
    m_ref[...] = acc * dcol
    dv_ref[...] = dcol

  return pl.pallas_call(
      body,
      grid=(n_pad // bm,),
      in_specs=[
          pl.BlockSpec((bm, nfeat), lambda i: (i, 0)),
          pl.BlockSpec((nfeat, h0), lambda i: (0, 0)),
          pl.BlockSpec((NC, bm), lambda i: (0, i)),
      ],
      out_specs=[
          pl.BlockSpec((bm, h0), lambda i: (i, 0)),
          pl.BlockSpec((bm, 1), lambda i: (i, 0)),
      ],
      out_shape=[
          jax.ShapeDtypeStruct((n_pad, h0), jnp.float32),
          jax.ShapeDtypeStruct((n_pad, 1), jnp.float32),
      ],
  )(x_p, w0, degp)


def _tc_mid(p0, p1, m1s, dinv, b0r, w1, n_pad, bm=1024):
  h0 = m1s.shape[1]
  ncls = w1.shape[1]

  def body(p0_ref, p1_ref, m_ref, dv_ref, b_ref, w_ref, o_ref):
    t = (p0_ref[...] + p1_ref[...] + m_ref[...]) * dv_ref[...] + b_ref[...]
    h1 = jnp.maximum(t, 0.0)
    o_ref[...] = jnp.dot(
        h1, w_ref[...], preferred_element_type=jnp.float32) * dv_ref[...]

  return pl.pallas_call(
      body,
      grid=(n_pad // bm,),
      in_specs=[
          pl.BlockSpec((bm, h0), lambda i: (i, 0)),
          pl.BlockSpec((bm, h0), lambda i: (i, 0)),
          pl.BlockSpec((bm, h0), lambda i: (i, 0)),
          pl.BlockSpec((bm, 1), lambda i: (i, 0)),
          pl.BlockSpec((1, h0), lambda i: (0, 0)),
          pl.BlockSpec((h0, ncls), lambda i: (0, 0)),
      ],
      out_specs=pl.BlockSpec((bm, ncls), lambda i: (i, 0)),
      out_shape=jax.ShapeDtypeStruct((n_pad, ncls), jnp.float32),
  )(p0, p1, m1s, dinv, b0r, w1)


def _tc_out(q0, q1, m2s, dinv, b1r, n_pad, bm=1024):
  ncls = m2s.shape[1]

  def body(q0_ref, q1_ref, m_ref, dv_ref, b_ref, o_ref):
    t = (q0_ref[...] + q1_ref[...] + m_ref[...]) * dv_ref[...] + b_ref[...]
    z = jnp.maximum(t, 0.0)
    zmax = jnp.max(z, axis=1, keepdims=True)
    e = jnp.exp(z - zmax)
    ssum = jnp.sum(e, axis=1, keepdims=True)
    o_ref[...] = (z - zmax) - jnp.log(ssum)

  return pl.pallas_call(
      body,
      grid=(n_pad // bm,),
      in_specs=[
          pl.BlockSpec((bm, ncls), lambda i: (i, 0)),
          pl.BlockSpec((bm, ncls), lambda i: (i, 0)),
          pl.BlockSpec((bm, ncls), lambda i: (i, 0)),
          pl.BlockSpec((bm, 1), lambda i: (i, 0)),
          pl.BlockSpec((1, ncls), lambda i: (0, 0)),
      ],
      out_specs=pl.BlockSpec((bm, ncls), lambda i: (i, 0)),
      out_shape=jax.ShapeDtypeStruct((n_pad, ncls), jnp.float32),
  )(q0, q1, m2s, dinv, b1r)


def kernel(x, edge_index, W0, b0, W1, b1):
  n, nfeat = x.shape
  e = edge_index.shape[1]

  n_pad = -(-n // 2048) * 2048
  eq = NS * CHUNK * 2 * DI
  e_pad = -(-e // eq) * eq
  ct16 = e_pad // CHUNK // NS
  cpw0 = (ct16 * 75 // 100 // DI) * DI
  cpw0 = max(DI, min(ct16 - DI, cpw0))

  x_p = jnp.zeros((n_pad, nfeat), jnp.float32).at[:n].set(x)
  pad = jnp.full((e_pad - e,), n, jnp.int32)
  src = jnp.concatenate([edge_index[0], pad])
  dst = jnp.concatenate([edge_index[1], pad])

  degp = _sc_deg(dst, n_pad, e_pad, cpw0).reshape(NC, n_pad)
  m1s, dinv = _tc_mm1(x_p, W0, degp, n_pad)

  p = _sc_agg(m1s, src, dst, n_pad, e_pad, W0.shape[1], cpw0)
  p = p.reshape(NC, n_pad, -1)
  m2s = _tc_mid(p[0], p[1], m1s, dinv, b0.reshape(1, -1), W1, n_pad)

  q = _sc_agg(m2s, src, dst, n_pad, e_pad, W1.shape[1], cpw0)
  q = q.reshape(NC, n_pad, -1)
  out = _tc_out(q[0], q[1], m2s, dinv, b1.reshape(1, -1), n_pad)
  return out[:n]


# speedup vs baseline: 15.6895x; 1.0001x over previous
import functools

import jax
import jax.numpy as jnp
from jax import lax
from jax.experimental import pallas as pl
from jax.experimental.pallas import tpu as pltpu
from jax.experimental.pallas import tpu_sc as plsc

NC = 2
NS = 16
NW = NC * NS
CHUNK = 128
LANES = 16


def _sc_mesh():
  return plsc.VectorSubcoreMesh(core_axis_name="c", subcore_axis_name="s")


@functools.partial(jax.jit, static_argnums=(1, 2, 3))
def _sc_deg(dst, n_pad, e_pad, cpw0):
  ct16 = e_pad // CHUNK // NS
  cpw1 = ct16 - cpw0
  cpw = max(cpw0, cpw1)
  rpt = n_pad // NS

  @functools.partial(
      pl.kernel,
      mesh=_sc_mesh(),
      compiler_params=pltpu.CompilerParams(use_tc_tiling_on_sc=False),
      out_type=jax.ShapeDtypeStruct((NC * n_pad,), jnp.float32),
      scratch_types=[
          pltpu.VMEM((cpw, CHUNK), jnp.int32),
          pltpu.VMEM((CHUNK,), jnp.float32),
          pltpu.VMEM((rpt,), jnp.float32),
          pltpu.VMEM_SHARED((n_pad,), jnp.float32),
          pltpu.SemaphoreType.DMA,
          pltpu.SemaphoreType.DMA,
      ],
  )
  def deg_kernel(dst_hbm, out_hbm, didx_v, ones_v, bounce_v, acc_sh,
                 sem_i, sem_s):
    c = lax.axis_index("c")
    s = lax.axis_index("s")
    cpw_c = jnp.where(c == 0, cpw0, cpw1)
    base = jnp.where(c == 0, s * cpw0, NS * cpw0 + s * cpw1) * CHUNK

    def ld(i, _):
      pltpu.async_copy(
          dst_hbm.at[pl.ds(base + i * CHUNK, CHUNK)], didx_v.at[i], sem_i)
      return 0
    lax.fori_loop(0, cpw_c, ld, 0)

    def init_ones(i, _):
      ones_v[pl.ds(i * LANES, LANES)] = jnp.full((LANES,), 1.0, jnp.float32)
      return 0
    lax.fori_loop(0, CHUNK // LANES, init_ones, 0)

    def init_zero(i, _):
      bounce_v[pl.ds(i * LANES, LANES)] = jnp.zeros((LANES,), jnp.float32)
      return 0
    lax.fori_loop(0, rpt // LANES, init_zero, 0)

    pltpu.sync_copy(bounce_v, acc_sh.at[pl.ds(s * rpt, rpt)])

    def ld_drain(i, _):
      pltpu.make_async_copy(
          dst_hbm.at[pl.ds(0, CHUNK)], didx_v.at[0], sem_i).wait()
      return 0
    lax.fori_loop(0, cpw_c, ld_drain, 0)
    plsc.subcore_barrier()

    def body(i, _):
      pltpu.async_copy(ones_v, acc_sh.at[didx_v.at[i]], sem_s, add=True)
      return 0
    lax.fori_loop(0, cpw_c, body, 0)

    def drain(i, _):
      pltpu.make_async_copy(ones_v, acc_sh.at[didx_v.at[0]], sem_s).wait()
      return 0
    lax.fori_loop(0, cpw_c, drain, 0)

    plsc.subcore_barrier()
    pltpu.sync_copy(acc_sh.at[pl.ds(s * rpt, rpt)], bounce_v)
    pltpu.sync_copy(bounce_v, out_hbm.at[pl.ds(c * n_pad + s * rpt, rpt)])

  return deg_kernel(dst)


RING = 5
PF = 3
RP = RING - PF
DI = 10
KI = 8


@functools.partial(jax.jit, static_argnums=(3, 4, 5, 6))
def _sc_agg(table, src, dst, n_pad, e_pad, h, cpw0):
  ct16 = e_pad // CHUNK // NS
  cpw1 = ct16 - cpw0
  rpt = n_pad // NS
  zr = 32
  nz = rpt // zr

  @functools.partial(
      pl.kernel,
      mesh=_sc_mesh(),
      compiler_params=pltpu.CompilerParams(use_tc_tiling_on_sc=False),
      out_type=jax.ShapeDtypeStruct((NC * n_pad, h), jnp.float32),
      scratch_types=[
          pltpu.VMEM((DI, CHUNK), jnp.int32),
          pltpu.VMEM((DI, CHUNK), jnp.int32),
          pltpu.VMEM((RING, CHUNK, h), jnp.float32),
          pltpu.VMEM((zr, h), jnp.float32),
          pltpu.VMEM_SHARED((n_pad, h), jnp.float32),
          pltpu.SemaphoreType.DMA((DI,)),
          pltpu.SemaphoreType.DMA((RING,)),
          pltpu.SemaphoreType.DMA((RING,)),
      ],
  )
  def agg_kernel(tbl_hbm, src_hbm, dst_hbm, out_hbm,
                 sidx_v, didx_v, rows_v, bounce_v, acc_sh,
                 sem_i, sem_g, sem_s):
    c = lax.axis_index("c")
    s = lax.axis_index("s")
    cpw_c = jnp.where(c == 0, cpw0, cpw1)
    base = jnp.where(c == 0, s * cpw0, NS * cpw0 + s * cpw1) * CHUNK

    def ld(i, sl):
      pltpu.async_copy(
          src_hbm.at[pl.ds(base + i * CHUNK, CHUNK)], sidx_v.at[sl],
          sem_i.at[sl])
      pltpu.async_copy(
          dst_hbm.at[pl.ds(base + i * CHUNK, CHUNK)], didx_v.at[sl],
          sem_i.at[sl])

    def iwait(sl):
      pltpu.make_async_copy(
          src_hbm.at[pl.ds(0, CHUNK)], sidx_v.at[0], sem_i.at[sl]).wait()
      pltpu.make_async_copy(
          src_hbm.at[pl.ds(0, CHUNK)], didx_v.at[0], sem_i.at[sl]).wait()

    def gather(i, b, sl):
      del i
      pltpu.async_copy(tbl_hbm.at[sidx_v.at[sl]], rows_v.at[b], sem_g.at[b])

    def gwait(b):
      pltpu.make_async_copy(
          tbl_hbm.at[sidx_v.at[0]], rows_v.at[b], sem_g.at[b]).wait()

    def scatter(i, b, sl):
      del i
      pltpu.async_copy(
          rows_v.at[b], acc_sh.at[didx_v.at[sl]], sem_s.at[b], add=True)

    def swait(b):
      pltpu.make_async_copy(
          rows_v.at[b], acc_sh.at[didx_v.at[0]], sem_s.at[b]).wait()

    @pl.when(cpw_c > 0)
    def _work():
      for t in range(KI):
        ld(t, t)

      def init_zero(i, _):
        r = i // (h // LANES)
        j = i % (h // LANES)
        bounce_v[r, pl.ds(j * LANES, LANES)] = jnp.zeros(
            (LANES,), jnp.float32)
        return 0
      lax.fori_loop(0, zr * (h // LANES), init_zero, 0)

      def zero_acc(k, _):
        pltpu.sync_copy(bounce_v, acc_sh.at[pl.ds(s * rpt + k * zr, zr)])
        return 0
      lax.fori_loop(0, nz, zero_acc, 0)
      plsc.subcore_barrier()

      for t in range(PF):
        iwait(t)
        gather(t, t % RING, t)

      def handler(ibase, jj, t, load_ok, gather_ok, guard):
        i = ibase + t
        b = t % RING
        bp = (b + PF) % RING
        gwait(b)
        scatter(i, b, t)
        if guard:
          @pl.when(jj > 0)
          def _():
            swait(bp)
        else:
          swait(bp)
        if load_ok:
          ld(i + KI, (t + KI) % DI)
        if gather_ok:
          iwait((t + PF) % DI)
          gather(i + PF, bp, (t + PF) % DI)

      def group(jj, _):
        ibase = jj * DI
        for t in range(DI):
          handler(ibase, jj, t, True, True, t < RP)
        return 0
      lax.fori_loop(0, cpw_c // DI - 1, group, 0)

      ibase = cpw_c - DI
      for t in range(DI):
        handler(ibase, 1, t, t + KI < DI, t + PF < DI, False)
      for t in range(DI - RP, DI):
        swait(t % RING)

      plsc.subcore_barrier()

      def writeout(k, _):
        pltpu.sync_copy(acc_sh.at[pl.ds(s * rpt + k * zr, zr)], bounce_v)
        pltpu.sync_copy(
            bounce_v, out_hbm.at[pl.ds(c * n_pad + s * rpt + k * zr, zr)])
        return 0
      lax.fori_loop(0, nz, writeout, 0)

  return agg_kernel(table, src, dst)


def _tc_mm1(x_p, w0, degp, n_pad, bm=1024):
  nfeat = x_p.shape[1]
  h0 = w0.shape[1]

  def body(x_ref, w_ref, dg_ref, m_ref, dv_ref):
    dsum = dg_ref[0:1, :] + dg_ref[1:2, :] + 1.0
    dcol = jnp.transpose(lax.rsqrt(dsum), (1, 0))
    acc = jnp.dot(x_ref[...], w_ref[...], preferred_element_type=jnp.float32)
    m_ref[...] = acc * dcol
    dv_ref[...] = dcol

  return pl.pallas_call(
      body,
      grid=(n_pad // bm,),
      in_specs=[
          pl.BlockSpec((bm, nfeat), lambda i: (i, 0)),
          pl.BlockSpec((nfeat, h0), lambda i: (0, 0)),
          pl.BlockSpec((NC, bm), lambda i: (0, i)),
      ],
      out_specs=[
          pl.BlockSpec((bm, h0), lambda i: (i, 0)),
          pl.BlockSpec((bm, 1), lambda i: (i, 0)),
      ],
      out_shape=[
          jax.ShapeDtypeStruct((n_pad, h0), jnp.float32),
          jax.ShapeDtypeStruct((n_pad, 1), jnp.float32),
      ],
  )(x_p, w0, degp)


def _tc_mid(p0, p1, m1s, dinv, b0r, w1, n_pad, bm=1024):
  h0 = m1s.shape[1]
  ncls = w1.shape[1]

  def body(p0_ref, p1_ref, m_ref, dv_ref, b_ref, w_ref, o_ref):
    t = (p0_ref[...] + p1_ref[...] + m_ref[...]) * dv_ref[...] + b_ref[...]
    h1 = jnp.maximum(t, 0.0)
    o_ref[...] = jnp.dot(
        h1, w_ref[...], preferred_element_type=jnp.float32) * dv_ref[...]

  return pl.pallas_call(
      body,
      grid=(n_pad // bm,),
      in_specs=[
          pl.BlockSpec((bm, h0), lambda i: (i, 0)),
          pl.BlockSpec((bm, h0), lambda i: (i, 0)),
          pl.BlockSpec((bm, h0), lambda i: (i, 0)),
          pl.BlockSpec((bm, 1), lambda i: (i, 0)),
          pl.BlockSpec((1, h0), lambda i: (0, 0)),
          pl.BlockSpec((h0, ncls), lambda i: (0, 0)),
      ],
      out_specs=pl.BlockSpec((bm, ncls), lambda i: (i, 0)),
      out_shape=jax.ShapeDtypeStruct((n_pad, ncls), jnp.float32),
  )(p0, p1, m1s, dinv, b0r, w1)


def _tc_out(q0, q1, m2s, dinv, b1r, n_pad, bm=1024):
  ncls = m2s.shape[1]

  def body(q0_ref, q1_ref, m_ref, dv_ref, b_ref, o_ref):
    t = (q0_ref[...] + q1_ref[...] + m_ref[...]) * dv_ref[...] + b_ref[...]
    z = jnp.maximum(t, 0.0)
    zmax = jnp.max(z, axis=1, keepdims=True)
    e = jnp.exp(z - zmax)
    ssum = jnp.sum(e, axis=1, keepdims=True)
    o_ref[...] = (z - zmax) - jnp.log(ssum)

  return pl.pallas_call(
      body,
      grid=(n_pad // bm,),
      in_specs=[
          pl.BlockSpec((bm, ncls), lambda i: (i, 0)),
          pl.BlockSpec((bm, ncls), lambda i: (i, 0)),
          pl.BlockSpec((bm, ncls), lambda i: (i, 0)),
          pl.BlockSpec((bm, 1), lambda i: (i, 0)),
          pl.BlockSpec((1, ncls), lambda i: (0, 0)),
      ],
      out_specs=pl.BlockSpec((bm, ncls), lambda i: (i, 0)),
      out_shape=jax.ShapeDtypeStruct((n_pad, ncls), jnp.float32),
  )(q0, q1, m2s, dinv, b1r)


def kernel(x, edge_index, W0, b0, W1, b1):
  n, nfeat = x.shape
  e = edge_index.shape[1]

  n_pad = -(-n // 2048) * 2048
  eq = NS * CHUNK * 2 * DI
  e_pad = -(-e // eq) * eq
  ct16 = e_pad // CHUNK // NS
  cpw0 = (ct16 * 75 // 100 // DI) * DI
  cpw0 = max(DI, min(ct16 - DI, cpw0))

  x_p = jnp.zeros((n_pad, nfeat), jnp.float32).at[:n].set(x)
  pad = jnp.full((e_pad - e,), n, jnp.int32)
  src = jnp.concatenate([edge_index[0], pad])
  dst = jnp.concatenate([edge_index[1], pad])

  degp = _sc_deg(dst, n_pad, e_pad, cpw0).reshape(NC, n_pad)
  m1s, dinv = _tc_mm1(x_p, W0, degp, n_pad)

  p = _sc_agg(m1s, src, dst, n_pad, e_pad, W0.shape[1], cpw0)
  p = p.reshape(NC, n_pad, -1)
  m2s = _tc_mid(p[0], p[1], m1s, dinv, b0.reshape(1, -1), W1, n_pad)

  q = _sc_agg(m2s, src, dst, n_pad, e_pad, W1.shape[1], cpw0)
  q = q.reshape(NC, n_pad, -1)
  out = _tc_out(q[0], q[1], m2s, dinv, b1.reshape(1, -1), n_pad)
  return out[:n]
